# baseline (speedup 1.0000x reference)
import functools

import jax
import jax.numpy as jnp
import numpy as np
from jax import lax
from jax.experimental import pallas as pl
from jax.experimental.pallas import tpu as pltpu
from jax.experimental.pallas import tpu_sc as plsc

VOCAB = 100000
MAX_SEQ_LEN = 2048
D_MODEL = 768

B = 4
S = 2048
NW = 32
S_PER_W = S // NW
LANES = 16
VECS_PER_ROW = D_MODEL // LANES


def _pos_encoding(max_seq_len, d_model):
    pos = np.arange(max_seq_len, dtype=np.float32)[:, None]
    div = np.exp(
        np.arange(0, d_model, 2, dtype=np.float32) * (-np.log(10000.0) / d_model)
    )
    pe = np.zeros((max_seq_len, d_model), dtype=np.float32)
    pe[:, 0::2] = np.sin(pos * div)
    pe[:, 1::2] = np.cos(pos * div)
    return pe


_PE = _pos_encoding(MAX_SEQ_LEN, D_MODEL)


def _make_sc_call():
    mesh = plsc.VectorSubcoreMesh(core_axis_name="c", subcore_axis_name="s")

    @functools.partial(
        pl.kernel,
        mesh=mesh,
        out_type=jax.ShapeDtypeStruct((B, S, D_MODEL), jnp.float32),
        scratch_types=[
            pltpu.VMEM((B, S_PER_W), jnp.int32),
            pltpu.VMEM((S_PER_W, D_MODEL), jnp.float32),
            pltpu.VMEM((S_PER_W, D_MODEL), jnp.float32),
            pltpu.SemaphoreType.DMA,
        ],
    )
    def emb_kernel(xt_hbm, w_hbm, pe_hbm, out_hbm, idx_v, pe_v, rows_v, sem):
        wid = lax.axis_index("s") * 2 + lax.axis_index("c")
        seq_base = wid * S_PER_W
        pltpu.sync_copy(xt_hbm.at[wid], idx_v)
        pltpu.sync_copy(pe_hbm.at[wid], pe_v)
        for b in range(B):
            pltpu.async_copy(w_hbm.at[idx_v.at[b]], rows_v, sem).wait()

            def add_row(i, _):
                for j in range(VECS_PER_ROW):
                    sl = pl.ds(j * LANES, LANES)
                    rows_v[i, sl] = rows_v[i, sl] + pe_v[i, sl]
                return 0

            lax.fori_loop(0, S_PER_W, add_row, 0)
            pltpu.sync_copy(rows_v, out_hbm.at[b, pl.ds(seq_base, S_PER_W)])

    return emb_kernel


_SC_CALL = _make_sc_call()


def kernel(x, W):
    xt = x.astype(jnp.int32).reshape(B, NW, S_PER_W).transpose(1, 0, 2)
    pe = jnp.asarray(_PE).reshape(NW, S_PER_W, D_MODEL)
    return _SC_CALL(xt, W, pe)


# speedup vs baseline: 1.0090x; 1.0090x over previous
import functools

import jax
import jax.numpy as jnp
import numpy as np
from jax import lax
from jax.experimental import pallas as pl
from jax.experimental.pallas import tpu as pltpu
from jax.experimental.pallas import tpu_sc as plsc

VOCAB = 100000
MAX_SEQ_LEN = 2048
D_MODEL = 768

B = 4
S = 2048
NW = 32
S_PER_W = S // NW
LANES = 16
VECS_PER_ROW = D_MODEL // LANES


def _pos_encoding(max_seq_len, d_model):
    pos = np.arange(max_seq_len, dtype=np.float32)[:, None]
    div = np.exp(
        np.arange(0, d_model, 2, dtype=np.float32) * (-np.log(10000.0) / d_model)
    )
    pe = np.zeros((max_seq_len, d_model), dtype=np.float32)
    pe[:, 0::2] = np.sin(pos * div)
    pe[:, 1::2] = np.cos(pos * div)
    return pe


_PE = _pos_encoding(MAX_SEQ_LEN, D_MODEL)


CH = 32
NCH = (B * S_PER_W) // CH
NBUF = 3


def _make_sc_call():
    mesh = plsc.VectorSubcoreMesh(core_axis_name="c", subcore_axis_name="s")

    @functools.partial(
        pl.kernel,
        mesh=mesh,
        out_type=jax.ShapeDtypeStruct((B, S, D_MODEL), jnp.float32),
        scratch_types=[
            pltpu.VMEM((B, S_PER_W), jnp.int32),
            pltpu.VMEM((S_PER_W, D_MODEL), jnp.float32),
            pltpu.VMEM((CH, D_MODEL), jnp.float32),
            pltpu.VMEM((CH, D_MODEL), jnp.float32),
            pltpu.VMEM((CH, D_MODEL), jnp.float32),
            pltpu.SemaphoreType.DMA,
            pltpu.SemaphoreType.DMA,
            pltpu.SemaphoreType.DMA,
            pltpu.SemaphoreType.DMA,
            pltpu.SemaphoreType.DMA,
            pltpu.SemaphoreType.DMA,
            pltpu.SemaphoreType.DMA,
        ],
    )
    def emb_kernel(xt_hbm, w_hbm, pe_hbm, out_hbm, idx_v, pe_v,
                   r0, r1, r2, g0, g1, g2, o0, o1, o2, psem):
        rows = (r0, r1, r2)
        gsem = (g0, g1, g2)
        osem = (o0, o1, o2)
        wid = lax.axis_index("s") * 2 + lax.axis_index("c")
        seq_base = wid * S_PER_W

        pe_copy = pltpu.async_copy(pe_hbm.at[wid], pe_v, psem)
        pltpu.sync_copy(xt_hbm.at[wid], idx_v)

        def fire_gather(c):
            b, h = divmod(c, S_PER_W // CH)
            s = c % NBUF
            return pltpu.async_copy(
                w_hbm.at[idx_v.at[b, pl.ds(h * CH, CH)]], rows[s], gsem[s])

        gathers = {}
        outs = {}
        for c in range(min(NBUF, NCH)):
            gathers[c] = fire_gather(c)
        pe_copy.wait()

        for c in range(NCH):
            b, h = divmod(c, S_PER_W // CH)
            s = c % NBUF
            gathers[c].wait()

            def add_row(i, _, _s=s, _h=h):
                for j in range(VECS_PER_ROW):
                    sl = pl.ds(j * LANES, LANES)
                    plsc.addupdate(rows[_s].at[i, sl], pe_v[_h * CH + i, sl])
                return 0

            lax.fori_loop(0, CH, add_row, 0)

            nxt = c + NBUF - 1
            if NBUF <= nxt < NCH:
                outs[c - 1].wait()
                gathers[nxt] = fire_gather(nxt)
            outs[c] = pltpu.async_copy(
                rows[s], out_hbm.at[b, pl.ds(seq_base + h * CH, CH)], osem[s])

        for c in range(max(0, NCH - NBUF), NCH):
            outs[c].wait()

    return emb_kernel


_SC_CALL = _make_sc_call()


def kernel(x, W):
    xt = x.astype(jnp.int32).reshape(B, NW, S_PER_W).transpose(1, 0, 2)
    pe = jnp.asarray(_PE).reshape(NW, S_PER_W, D_MODEL)
    return _SC_CALL(xt, W, pe)
